# baseline (speedup 1.0000x reference)
import functools

import jax
import jax.numpy as jnp
from jax import lax
from jax.experimental import pallas as pl
from jax.experimental.pallas import tpu as pltpu
from jax.experimental.pallas import tpu_sc as plsc

N_EDGES = 320000
VOCAB = 100000
EMB_DIM = 48
HIDDEN = 128
LANES = 16

NC, NS = 2, 16
NW = NC * NS
ROWS_PER_W = N_EDGES // NW
CHUNK = 400
NCHUNK = ROWS_PER_W // CHUNK


def _gather_sum_body(e_hbm, t0, t1, t2, t3, s_hbm,
                     ec_v, i0, i1, i2, i3, rows_v, sem):
    wid = lax.axis_index("s") * NC + lax.axis_index("c")
    base = wid * ROWS_PER_W
    idx_bufs = (i0, i1, i2, i3)

    def chunk_body(c, carry):
        off = c * CHUNK
        pltpu.sync_copy(e_hbm.at[pl.ds(base + off, CHUNK)], ec_v)

        def g_body(g, carry2):
            rows = g * LANES + lax.iota(jnp.int32, LANES)
            for f in range(4):
                cols = jnp.full((LANES,), f, jnp.int32)
                idx_bufs[f][pl.ds(g * LANES, LANES)] = plsc.load_gather(
                    ec_v, [rows, cols])
            return carry2

        lax.fori_loop(0, CHUNK // LANES, g_body, 0, unroll=2)

        cps = [
            pltpu.async_copy(tbl.at[idx_bufs[f]], rows_v.at[f], sem)
            for f, tbl in enumerate((t0, t1, t2, t3))
        ]
        for cp in cps:
            cp.wait()

        def row_body(r, carry2):
            for j in range(EMB_DIM // LANES):
                sl = pl.ds(j * LANES, LANES)
                acc = (rows_v[0, r, sl] + rows_v[1, r, sl]
                       + rows_v[2, r, sl] + rows_v[3, r, sl])
                rows_v[0, r, sl] = acc
            return carry2

        lax.fori_loop(0, CHUNK, row_body, 0, unroll=2)
        pltpu.sync_copy(rows_v.at[0], s_hbm.at[pl.ds(base + off, CHUNK)])
        return carry

    lax.fori_loop(0, NCHUNK, chunk_body, 0)


_gather_sum_cache = []


def _gather_sum(*args):
    if not _gather_sum_cache:
        _gather_sum_cache.append(functools.partial(
            pl.kernel,
            out_type=jax.ShapeDtypeStruct((N_EDGES, EMB_DIM), jnp.float32),
            mesh=plsc.VectorSubcoreMesh(core_axis_name="c",
                                        subcore_axis_name="s",
                                        num_cores=NC, num_subcores=NS),
            scratch_types=[
                pltpu.VMEM((CHUNK, 4), jnp.int32),
                pltpu.VMEM((CHUNK,), jnp.int32),
                pltpu.VMEM((CHUNK,), jnp.int32),
                pltpu.VMEM((CHUNK,), jnp.int32),
                pltpu.VMEM((CHUNK,), jnp.int32),
                pltpu.VMEM((4, CHUNK, EMB_DIM), jnp.float32),
                pltpu.SemaphoreType.DMA,
            ],
            compiler_params=pltpu.CompilerParams(use_tc_tiling_on_sc=False,
                                                 needs_layout_passes=False),
        )(_gather_sum_body))
    return _gather_sum_cache[0](*args)


TC_BLK = 3200


def _proj_body(s_ref, w_ref, b_ref, o_ref):
    h = jnp.dot(s_ref[...], w_ref[...],
                preferred_element_type=jnp.float32) + b_ref[...]
    o_ref[...] = 0.5 * h * (1.0 + lax.erf(h * 0.7071067811865476))


_proj = pl.pallas_call(
    _proj_body,
    grid=(N_EDGES // TC_BLK,),
    in_specs=[
        pl.BlockSpec((TC_BLK, EMB_DIM), lambda i: (i, 0)),
        pl.BlockSpec((EMB_DIM, HIDDEN), lambda i: (0, 0)),
        pl.BlockSpec((1, HIDDEN), lambda i: (0, 0)),
    ],
    out_specs=pl.BlockSpec((TC_BLK, HIDDEN), lambda i: (i, 0)),
    out_shape=jax.ShapeDtypeStruct((N_EDGES, HIDDEN), jnp.float32),
)


def kernel(e, emb0, emb1, emb2, emb3, W, b):
    e32 = e.astype(jnp.int32)
    s = _gather_sum(e32, emb0, emb1, emb2, emb3)
    return _proj(s, W, b.reshape(1, HIDDEN))


# speedup vs baseline: 1.5970x; 1.5970x over previous
import functools

import jax
import jax.numpy as jnp
from jax import lax
from jax.experimental import pallas as pl
from jax.experimental.pallas import tpu as pltpu
from jax.experimental.pallas import tpu_sc as plsc

N_EDGES = 320000
VOCAB = 100000
EMB_DIM = 48
HIDDEN = 128
LANES = 16

NC, NS = 2, 16
NW = NC * NS
ROWS_PER_W = N_EDGES // NW
CHUNK = 200
NCHUNK = ROWS_PER_W // CHUNK
NPAIR = NCHUNK // 2


def _gather_sum_body(e0, e1, e2, e3, t0, t1, t2, t3, s_hbm,
                     i0, i1, i2, i3, rows_a, rows_b, sem_a, sem_b):
    wid = lax.axis_index("s") * NC + lax.axis_index("c")
    base = wid * ROWS_PER_W
    idx_bufs = (i0, i1, i2, i3)
    tables = (t0, t1, t2, t3)

    for f, e_f in enumerate((e0, e1, e2, e3)):
        pltpu.sync_copy(e_f.at[pl.ds(base, ROWS_PER_W)], idx_bufs[f])

    def fire(c, buf, sem):
        off = c * CHUNK
        for f in range(4):
            pltpu.async_copy(tables[f].at[idx_bufs[f].at[pl.ds(off, CHUNK)]],
                             buf.at[f], sem)

    def drain(buf, sem):
        for f in range(4):
            pltpu.make_async_copy(tables[f].at[idx_bufs[f].at[pl.ds(0, CHUNK)]],
                                  buf.at[f], sem).wait()

    def sum_and_out(c, buf):
        def row_body(r, carry):
            for j in range(EMB_DIM // LANES):
                sl = pl.ds(j * LANES, LANES)
                acc = (buf[0, r, sl] + buf[1, r, sl]
                       + buf[2, r, sl] + buf[3, r, sl])
                buf[0, r, sl] = acc
            return carry

        lax.fori_loop(0, CHUNK, row_body, 0, unroll=2)
        pltpu.sync_copy(buf.at[0], s_hbm.at[pl.ds(base + c * CHUNK, CHUNK)])

    fire(0, rows_a, sem_a)

    def pair_body(cp, carry):
        c0 = cp * 2
        fire(c0 + 1, rows_b, sem_b)
        drain(rows_a, sem_a)
        sum_and_out(c0, rows_a)

        @pl.when(cp < NPAIR - 1)
        def _():
            fire(c0 + 2, rows_a, sem_a)

        drain(rows_b, sem_b)
        sum_and_out(c0 + 1, rows_b)
        return carry

    lax.fori_loop(0, NPAIR, pair_body, 0)


_gather_sum_cache = []


def _gather_sum(*args):
    if not _gather_sum_cache:
        _gather_sum_cache.append(functools.partial(
            pl.kernel,
            out_type=jax.ShapeDtypeStruct((N_EDGES, EMB_DIM), jnp.float32),
            mesh=plsc.VectorSubcoreMesh(core_axis_name="c",
                                        subcore_axis_name="s",
                                        num_cores=NC, num_subcores=NS),
            scratch_types=[
                pltpu.VMEM((ROWS_PER_W,), jnp.int32),
                pltpu.VMEM((ROWS_PER_W,), jnp.int32),
                pltpu.VMEM((ROWS_PER_W,), jnp.int32),
                pltpu.VMEM((ROWS_PER_W,), jnp.int32),
                pltpu.VMEM((4, CHUNK, EMB_DIM), jnp.float32),
                pltpu.VMEM((4, CHUNK, EMB_DIM), jnp.float32),
                pltpu.SemaphoreType.DMA,
                pltpu.SemaphoreType.DMA,
            ],
            compiler_params=pltpu.CompilerParams(use_tc_tiling_on_sc=False),
        )(_gather_sum_body))
    return _gather_sum_cache[0](*args)


TC_BLK = 3200


def _proj_body(s_ref, w_ref, b_ref, o_ref):
    h = jnp.dot(s_ref[...], w_ref[...],
                preferred_element_type=jnp.float32) + b_ref[...]
    o_ref[...] = 0.5 * h * (1.0 + lax.erf(h * 0.7071067811865476))


_proj = pl.pallas_call(
    _proj_body,
    grid=(N_EDGES // TC_BLK,),
    in_specs=[
        pl.BlockSpec((TC_BLK, EMB_DIM), lambda i: (i, 0)),
        pl.BlockSpec((EMB_DIM, HIDDEN), lambda i: (0, 0)),
        pl.BlockSpec((1, HIDDEN), lambda i: (0, 0)),
    ],
    out_specs=pl.BlockSpec((TC_BLK, HIDDEN), lambda i: (i, 0)),
    out_shape=jax.ShapeDtypeStruct((N_EDGES, HIDDEN), jnp.float32),
)


def kernel(e, emb0, emb1, emb2, emb3, W, b):
    e32 = e.astype(jnp.int32)
    cols = [e32[:, f] for f in range(4)]
    s = _gather_sum(*cols, emb0, emb1, emb2, emb3)
    return _proj(s, W, b.reshape(1, HIDDEN))


# speedup vs baseline: 1.9647x; 1.2302x over previous
import functools

import jax
import jax.numpy as jnp
from jax import lax
from jax.experimental import pallas as pl
from jax.experimental.pallas import tpu as pltpu
from jax.experimental.pallas import tpu_sc as plsc

N_EDGES = 320000
VOCAB = 100000
EMB_DIM = 48
HIDDEN = 128
LANES = 16

NC, NS = 2, 16
NW = NC * NS
ROWS_PER_W = N_EDGES // NW
CHUNK = 200
NCHUNK = ROWS_PER_W // CHUNK
NPAIR = NCHUNK // 2


def _gather_sum_body(e0, e1, e2, e3, t0, t1, t2, t3, s_hbm,
                     i0, i1, i2, i3, rows_a, rows_b, sem_a, sem_b):
    wid = lax.axis_index("s") * NC + lax.axis_index("c")
    base = wid * ROWS_PER_W
    idx_bufs = (i0, i1, i2, i3)
    tables = (t0, t1, t2, t3)

    for f, e_f in enumerate((e0, e1, e2, e3)):
        pltpu.sync_copy(e_f.at[pl.ds(base, ROWS_PER_W)], idx_bufs[f])

    def fire(c, buf, sem):
        off = c * CHUNK
        for f in range(4):
            pltpu.async_copy(tables[f].at[idx_bufs[f].at[pl.ds(off, CHUNK)]],
                             buf.at[f], sem)

    def drain(buf, sem):
        for f in range(4):
            pltpu.make_async_copy(tables[f].at[idx_bufs[f].at[pl.ds(0, CHUNK)]],
                                  buf.at[f], sem).wait()

    def sum_and_out(c, buf):
        def row_body(r, carry):
            for j in range(EMB_DIM // LANES):
                sl = pl.ds(j * LANES, LANES)
                acc = (buf[0, r, sl] + buf[1, r, sl]
                       + buf[2, r, sl] + buf[3, r, sl])
                buf[0, r, sl] = acc
            return carry

        lax.fori_loop(0, CHUNK, row_body, 0, unroll=2)
        pltpu.sync_copy(buf.at[0],
                        s_hbm.at[pl.ds(base + c * CHUNK, CHUNK),
                                 pl.ds(0, EMB_DIM)])

    fire(0, rows_a, sem_a)

    def pair_body(cp, carry):
        c0 = cp * 2
        fire(c0 + 1, rows_b, sem_b)
        drain(rows_a, sem_a)
        sum_and_out(c0, rows_a)

        @pl.when(cp < NPAIR - 1)
        def _():
            fire(c0 + 2, rows_a, sem_a)

        drain(rows_b, sem_b)
        sum_and_out(c0 + 1, rows_b)
        return carry

    lax.fori_loop(0, NPAIR, pair_body, 0)


_gather_sum_cache = []


def _gather_sum(*args):
    if not _gather_sum_cache:
        _gather_sum_cache.append(functools.partial(
            pl.kernel,
            out_type=jax.ShapeDtypeStruct((N_EDGES, HIDDEN), jnp.float32),
            mesh=plsc.VectorSubcoreMesh(core_axis_name="c",
                                        subcore_axis_name="s",
                                        num_cores=NC, num_subcores=NS),
            scratch_types=[
                pltpu.VMEM((ROWS_PER_W,), jnp.int32),
                pltpu.VMEM((ROWS_PER_W,), jnp.int32),
                pltpu.VMEM((ROWS_PER_W,), jnp.int32),
                pltpu.VMEM((ROWS_PER_W,), jnp.int32),
                pltpu.VMEM((4, CHUNK, EMB_DIM), jnp.float32),
                pltpu.VMEM((4, CHUNK, EMB_DIM), jnp.float32),
                pltpu.SemaphoreType.DMA,
                pltpu.SemaphoreType.DMA,
            ],
            compiler_params=pltpu.CompilerParams(use_tc_tiling_on_sc=False),
        )(_gather_sum_body))
    return _gather_sum_cache[0](*args)


TC_BLK = 3200


def _proj_body(s_ref, w_ref, b_ref, o_ref):
    h = jnp.dot(s_ref[:, :EMB_DIM], w_ref[...],
                preferred_element_type=jnp.float32) + b_ref[...]
    o_ref[...] = 0.5 * h * (1.0 + lax.erf(h * 0.7071067811865476))


_proj = pl.pallas_call(
    _proj_body,
    grid=(N_EDGES // TC_BLK,),
    in_specs=[
        pl.BlockSpec((TC_BLK, HIDDEN), lambda i: (i, 0)),
        pl.BlockSpec((EMB_DIM, HIDDEN), lambda i: (0, 0)),
        pl.BlockSpec((1, HIDDEN), lambda i: (0, 0)),
    ],
    out_specs=pl.BlockSpec((TC_BLK, HIDDEN), lambda i: (i, 0)),
    out_shape=jax.ShapeDtypeStruct((N_EDGES, HIDDEN), jnp.float32),
)


def kernel(e, emb0, emb1, emb2, emb3, W, b):
    e32 = e.astype(jnp.int32)
    cols = [e32[:, f] for f in range(4)]
    s = _gather_sum(*cols, emb0, emb1, emb2, emb3)
    return _proj(s, W, b.reshape(1, HIDDEN))
